# baseline (speedup 1.0000x reference)
import functools

import jax
import jax.numpy as jnp
from jax import lax
from jax.experimental import pallas as pl
from jax.experimental.pallas import tpu as pltpu
from jax.experimental.pallas import tpu_sc as plsc

DIM = 64
LANES = 16
CHUNK = 128


@functools.cache
def _build_sc_kernel(n_tokens: int, vocab: int):
    info = plsc.get_sparse_core_info()
    n_workers = info.num_cores * info.num_subcores
    per_worker = n_tokens // n_workers
    n_chunks = per_worker // CHUNK
    assert per_worker * n_workers == n_tokens
    assert n_chunks * CHUNK == per_worker

    mesh = plsc.VectorSubcoreMesh(core_axis_name="c", subcore_axis_name="s")

    @functools.partial(
        pl.kernel,
        mesh=mesh,
        out_type=jax.ShapeDtypeStruct((n_tokens, DIM), jnp.float32),
        compiler_params=pltpu.CompilerParams(use_tc_tiling_on_sc=False),
        scratch_types=[
            pltpu.VMEM((CHUNK,), jnp.int32),
            pltpu.VMEM((CHUNK,), jnp.float32),
            pltpu.VMEM((CHUNK, DIM), jnp.float32),
            pltpu.VMEM((CHUNK, DIM), jnp.float32),
            pltpu.VMEM((CHUNK, DIM), jnp.float32),
            pltpu.SemaphoreType.DMA,
        ],
    )
    def sc_combine(x_hbm, ctx_hbm, base_hbm, super_hbm, out_hbm,
                   idx_v, ctx_v, b_v, s_v, o_v, sem):
        wid = lax.axis_index("s") * info.num_cores + lax.axis_index("c")
        w_base = wid * per_worker

        def chunk_body(g, carry):
            off = w_base + g * CHUNK
            pltpu.sync_copy(x_hbm.at[pl.ds(off, CHUNK)], idx_v)
            pltpu.sync_copy(ctx_hbm.at[pl.ds(off, CHUNK)], ctx_v)
            cb = pltpu.async_copy(base_hbm.at[idx_v], b_v, sem)
            cs = pltpu.async_copy(super_hbm.at[idx_v], s_v, sem)
            cb.wait()
            cs.wait()

            def group_body(tg, c2):
                t0 = tg * LANES
                cv16 = ctx_v[pl.ds(t0, LANES)]
                for j in range(LANES):
                    cb = lax.gather(
                        cv16,
                        jnp.full((LANES, 1), j, jnp.int32),
                        lax.GatherDimensionNumbers(
                            offset_dims=(), collapsed_slice_dims=(0,),
                            start_index_map=(0,)),
                        (1,),
                        mode=lax.GatherScatterMode.PROMISE_IN_BOUNDS)
                    t = t0 + j
                    for d in range(DIM // LANES):
                        sl = pl.ds(d * LANES, LANES)
                        o_v[t, sl] = b_v[t, sl] + cb * s_v[t, sl]
                return c2

            lax.fori_loop(0, CHUNK // LANES, group_body, 0)
            pltpu.sync_copy(o_v, out_hbm.at[pl.ds(off, CHUNK)])
            return carry

        lax.fori_loop(0, n_chunks, chunk_body, 0)

    return sc_combine


def kernel(x, context_vector, base_table, super_table):
    b, l = x.shape
    n_tokens = b * l
    xf = x.reshape(n_tokens).astype(jnp.int32)
    cf = context_vector.reshape(n_tokens)
    sc = _build_sc_kernel(n_tokens, base_table.shape[0])
    out = sc(xf, cf, base_table, super_table)
    return out.reshape(b, l, DIM)


# speedup vs baseline: 1.2861x; 1.2861x over previous
import functools

import jax
import jax.numpy as jnp
from jax import lax
from jax.experimental import pallas as pl
from jax.experimental.pallas import tpu as pltpu
from jax.experimental.pallas import tpu_sc as plsc

DIM = 64
LANES = 16
CHUNK = 128
NSLOT = 4


def _bcast_lane(v, j):
    return lax.gather(
        v,
        jnp.full((LANES, 1), j, jnp.int32),
        lax.GatherDimensionNumbers(
            offset_dims=(), collapsed_slice_dims=(0,), start_index_map=(0,)),
        (1,),
        mode=lax.GatherScatterMode.PROMISE_IN_BOUNDS)


@functools.cache
def _build_sc_kernel(n_tokens: int):
    info = plsc.get_sparse_core_info()
    n_workers = info.num_cores * info.num_subcores
    per_worker = n_tokens // n_workers
    n_chunks = per_worker // CHUNK
    n_iters = n_chunks // NSLOT
    assert per_worker * n_workers == n_tokens
    assert n_iters * NSLOT == n_chunks

    mesh = plsc.VectorSubcoreMesh(core_axis_name="c", subcore_axis_name="s")

    @functools.partial(
        pl.kernel,
        mesh=mesh,
        out_type=jax.ShapeDtypeStruct((n_tokens, DIM), jnp.float32),
        compiler_params=pltpu.CompilerParams(use_tc_tiling_on_sc=False),
        scratch_types=[
            pltpu.VMEM((per_worker,), jnp.int32),
            pltpu.VMEM((per_worker,), jnp.float32),
            pltpu.VMEM((NSLOT, CHUNK, DIM), jnp.float32),
            pltpu.VMEM((NSLOT, CHUNK, DIM), jnp.float32),
            pltpu.SemaphoreType.DMA((NSLOT,)),
            pltpu.SemaphoreType.DMA((NSLOT,)),
        ],
    )
    def sc_combine(x_hbm, ctx_hbm, base_hbm, super_hbm, out_hbm,
                   idx_all, ctx_all, s_v, o_v, gsem, osem):
        wid = lax.axis_index("s") * info.num_cores + lax.axis_index("c")
        w_base = wid * per_worker

        pltpu.sync_copy(x_hbm.at[pl.ds(w_base, per_worker)], idx_all)
        pltpu.sync_copy(ctx_hbm.at[pl.ds(w_base, per_worker)], ctx_all)

        def fire_gathers(c, k):
            idx_slice = idx_all.at[pl.ds(c * CHUNK, CHUNK)]
            pltpu.async_copy(base_hbm.at[idx_slice], o_v.at[k], gsem.at[k])
            pltpu.async_copy(super_hbm.at[idx_slice], s_v.at[k], gsem.at[k])

        def drain_gathers(c, k):
            idx_slice = idx_all.at[pl.ds(c * CHUNK, CHUNK)]
            pltpu.make_async_copy(
                base_hbm.at[idx_slice], o_v.at[k], gsem.at[k]).wait()
            pltpu.make_async_copy(
                super_hbm.at[idx_slice], s_v.at[k], gsem.at[k]).wait()

        def fire_scatter(c, k):
            pltpu.async_copy(
                o_v.at[k], out_hbm.at[pl.ds(w_base + c * CHUNK, CHUNK)],
                osem.at[k])

        def drain_scatter(c, k):
            pltpu.make_async_copy(
                o_v.at[k], out_hbm.at[pl.ds(w_base + c * CHUNK, CHUNK)],
                osem.at[k]).wait()

        def compute(c, k):
            s_ref = s_v.at[k]
            o_ref = o_v.at[k]
            goff = c * CHUNK

            def group(tg, carry):
                t0 = tg * LANES
                cv16 = ctx_all[pl.ds(goff + t0, LANES)]
                for j in range(LANES):
                    cb = _bcast_lane(cv16, j)
                    t = t0 + j
                    for d in range(DIM // LANES):
                        sl = pl.ds(d * LANES, LANES)
                        plsc.addupdate(o_ref.at[t, sl], cb * s_ref[t, sl])
                return carry

            lax.fori_loop(0, CHUNK // LANES, group, 0)

        for k in range(NSLOT - 1):
            fire_gathers(k, k)

        def iter_body(q, carry):
            c0 = q * NSLOT
            for k in range(NSLOT):
                c = c0 + k
                drain_gathers(c, k)
                compute(c, k)
                fire_scatter(c, k)
                kn = (k + NSLOT - 1) % NSLOT
                if k == 0:
                    @pl.when(q > 0)
                    def _():
                        drain_scatter(c - 1, kn)
                    fire_gathers(c + NSLOT - 1, kn)
                else:
                    drain_scatter(c - 1, kn)
                    @pl.when(q < n_iters - 1)
                    def _():
                        fire_gathers(c + NSLOT - 1, kn)
            return carry

        lax.fori_loop(0, n_iters, iter_body, 0)
        drain_scatter(n_chunks - 1, (n_chunks - 1) % NSLOT)

    return sc_combine


def kernel(x, context_vector, base_table, super_table):
    b, l = x.shape
    n_tokens = b * l
    xf = x.reshape(n_tokens).astype(jnp.int32)
    cf = context_vector.reshape(n_tokens)
    sc = _build_sc_kernel(n_tokens)
    out = sc(xf, cf, base_table, super_table)
    return out.reshape(b, l, DIM)
